# Initial kernel scaffold; baseline (speedup 1.0000x reference)
import jax
import jax.numpy as jnp
from jax.experimental import pallas as pl


def kernel(spike_input, weights, spike_threshold):
    raise NotImplementedError("write your pallas kernel here")


# speedup vs baseline: 31.3328x; 31.3328x over previous
import functools

import jax
import jax.numpy as jnp
from jax import lax
from jax.experimental import pallas as pl
from jax.experimental.pallas import tpu as pltpu

MAP_H, MAP_W = 32, 32
INPUT_DIM = 256
BATCH = 256
LR = 0.1
RADIUS = 2.0
NCELLS = MAP_H * MAP_W


def _som_body(wf_ref, x_ref, a_ref, thr_ref, out_ref):
    wf = wf_ref[...]
    x = x_ref[...]
    wn = jnp.sum(wf * wf, axis=1, keepdims=True)
    xw = lax.dot_general(wf, x, (((1,), (1,)), ((), ())),
                         preferred_element_type=jnp.float32,
                         precision=lax.Precision.HIGHEST)
    scores = wn - 2.0 * xw
    m = jnp.min(scores, axis=0, keepdims=True)
    iota_c = lax.broadcasted_iota(jnp.int32, (NCELLS, BATCH), 0)
    bmu = jnp.min(jnp.where(scores == m, iota_c, 2**30),
                  axis=0, keepdims=True)
    by = bmu // MAP_W
    bx = bmu % MAP_W
    iota_y = lax.broadcasted_iota(jnp.int32, (MAP_H, BATCH), 0)
    oy = (iota_y == by).astype(jnp.float32)
    ox = (iota_y == bx).astype(jnp.float32)
    c = lax.dot_general(oy, ox, (((1,), (1,)), ((), ())),
                        preferred_element_type=jnp.float32,
                        precision=lax.Precision.HIGHEST)
    a_tab = a_ref[...]
    ca = lax.dot_general(c, a_tab, (((1,), (0,)), ((), ())),
                         preferred_element_type=jnp.float32,
                         precision=lax.Precision.HIGHEST)
    mp = 0.5 * lax.dot_general(a_tab, ca, (((1,), (0,)), ((), ())),
                               preferred_element_type=jnp.float32,
                               precision=lax.Precision.HIGHEST)
    out_ref[...] = (mp > thr_ref[...]).astype(jnp.float32)


@jax.jit
def kernel(spike_input, weights, spike_threshold):
    wf = weights.reshape(NCELLS, INPUT_DIM)
    u = jnp.arange(MAP_H, dtype=jnp.float32)
    d = u[:, None] - u[None, :]
    a_tab = jnp.exp(-(d * d) / (2.0 * RADIUS * RADIUS))
    return pl.pallas_call(
        _som_body,
        out_shape=jax.ShapeDtypeStruct((MAP_H, MAP_W), jnp.float32),
    )(wf, spike_input, a_tab, spike_threshold)
